# baseline (speedup 1.0000x reference)
import functools

import jax
import jax.numpy as jnp
from jax import lax
from jax.experimental import pallas as pl
from jax.experimental.pallas import tpu as pltpu
from jax.experimental.pallas import tpu_sc as plsc

_NUM_CORES = 2
_NUM_SUBCORES = 16
_NUM_WORKERS = _NUM_CORES * _NUM_SUBCORES
_LANES = 16


@functools.lru_cache(maxsize=None)
def _make_lookup(batch_size: int, vocab: int, dim: int):
    assert batch_size % (_NUM_WORKERS * _LANES) == 0
    assert dim & (dim - 1) == 0 and dim <= _LANES
    b_per_w = batch_size // _NUM_WORKERS
    out_per_w = b_per_w * dim
    n_groups = out_per_w // _LANES

    mesh = plsc.VectorSubcoreMesh(core_axis_name="c", subcore_axis_name="s")

    @functools.partial(
        pl.kernel,
        mesh=mesh,
        out_type=jax.ShapeDtypeStruct((batch_size, dim), jnp.float32),
        scratch_types=[
            pltpu.VMEM((b_per_w,), jnp.int32),
            pltpu.VMEM((vocab, dim), jnp.float32),
            pltpu.VMEM((b_per_w, dim), jnp.float32),
        ],
        compiler_params=pltpu.CompilerParams(needs_layout_passes=False),
    )
    def lookup(label_hbm, table_hbm, out_hbm, idx_v, table_v, out_v):
        wid = lax.axis_index("s") * _NUM_CORES + lax.axis_index("c")
        base = wid * b_per_w
        pltpu.sync_copy(table_hbm, table_v)
        pltpu.sync_copy(label_hbm.at[pl.ds(base, b_per_w)], idx_v)
        lane = lax.iota(jnp.int32, _LANES)
        shift = dim.bit_length() - 1
        row0 = lax.shift_right_logical(lane, shift)
        col = lax.bitwise_and(lane, dim - 1)
        rows_per_group = _LANES // dim

        @plsc.parallel_loop(0, n_groups, 1, unroll=16)
        def _group(g):
            row = row0 + g * rows_per_group
            lab = plsc.load_gather(idx_v, [row])
            val = plsc.load_gather(table_v, [lab, col])
            plsc.store_scatter(out_v, [row, col], val)

        pltpu.sync_copy(out_v, out_hbm.at[pl.ds(base, b_per_w)])

    return lookup


def kernel(batch, label, table):
    del batch
    (batch_size,) = label.shape
    vocab, dim = table.shape
    return _make_lookup(batch_size, vocab, dim)(label, table)
